# baseline (speedup 1.0000x reference)
import functools

import jax
import jax.numpy as jnp
from jax import lax
from jax.experimental import pallas as pl
from jax.experimental.pallas import tpu as pltpu
from jax.experimental.pallas import tpu_sc as plsc

N = 10000
D = 128
L = 16
NT = 32
PAD = 640

_f32 = jnp.float32
_i32 = jnp.int32


def _wid():
    return lax.axis_index("s") * 2 + lax.axis_index("c")


def _sread(ref, idx):
    return ref[pl.ds(idx, L)][0]


def _row_add(dst, dst_row, src, src_row):
    for j in range(D // L):
        v = src[src_row, pl.ds(L * j, L)]
        plsc.addupdate(dst.at[dst_row, pl.ds(L * j, L)], v)


def _zero_rows(buf, nrows):
    z = jnp.zeros((L,), _f32)

    def body(r, _):
        for j in range(D // L):
            buf[r, pl.ds(L * j, L)] = z
        return 0

    lax.fori_loop(0, nrows, body, 0)


def _mm_body(x_ref, w_ref, o_ref):
    o_ref[...] = jnp.dot(x_ref[...], w_ref[...], preferred_element_type=_f32)


def _tc_matmul(x, w):
    nnz = x.shape[0]
    br = 2048
    g = (nnz + br - 1) // br
    return pl.pallas_call(
        _mm_body,
        grid=(g,),
        in_specs=[
            pl.BlockSpec((br, D), lambda i: (i, 0)),
            pl.BlockSpec((D, D), lambda i: (0, 0)),
        ],
        out_specs=pl.BlockSpec((br, D), lambda i: (i, 0)),
        out_shape=jax.ShapeDtypeStruct((nnz, D), _f32),
    )(x, w)


def _node_mm_body(d_ref, p1_ref, p0_ref, x5_ref, w2, w3, w4, w5, w6, b_ref,
                  g0_ref, g1_ref):
    dd = d_ref[...]
    g0_ref[...] = (jnp.dot(dd, w2[...], preferred_element_type=_f32)
                   + jnp.dot(p0_ref[...], w5[...], preferred_element_type=_f32)
                   + jnp.dot(x5_ref[...], w6[...], preferred_element_type=_f32)
                   + b_ref[...])
    g1_ref[...] = (jnp.dot(dd, w3[...], preferred_element_type=_f32)
                   + jnp.dot(p1_ref[...], w4[...], preferred_element_type=_f32))


def _tc_node_matmul(diag, pool1, pool0, x5, w2, w3, w4, w5, w6, b2d):
    br = 1000
    g = N // br
    full = pl.BlockSpec((D, D), lambda i: (0, 0))
    blk = pl.BlockSpec((br, D), lambda i: (i, 0))
    return pl.pallas_call(
        _node_mm_body,
        grid=(g,),
        in_specs=[blk, blk, blk, blk, full, full, full, full, full,
                  pl.BlockSpec((1, D), lambda i: (0, 0))],
        out_specs=[blk, blk],
        out_shape=[jax.ShapeDtypeStruct((N, D), _f32),
                   jax.ShapeDtypeStruct((N, D), _f32)],
    )(diag, pool1, pool0, x5, w2, w3, w4, w5, w6, b2d)


def _make_pools_kernel(nnz):
    mesh = plsc.VectorSubcoreMesh(core_axis_name="c", subcore_axis_name="s")
    SN = 160
    NPART = (N // SN) * SN - (N // (SN * 2)) * SN * 2

    @functools.partial(
        pl.kernel,
        out_type=[jax.ShapeDtypeStruct((N, D), _f32),
                  jax.ShapeDtypeStruct((N, D), _f32),
                  jax.ShapeDtypeStruct((N, D), _f32)],
        mesh=mesh,
        scratch_types=[
            pltpu.VMEM((SN + 1, D), _f32),
            pltpu.VMEM((SN + 1, D), _f32),
            pltpu.VMEM((SN + 1, D), _f32),
            pltpu.VMEM((128, D), _f32),
            pltpu.VMEM((128, D), _f32),
            pltpu.VMEM((144,), _i32),
            pltpu.VMEM((144,), _i32),
            pltpu.VMEM((128,), _i32),
            pltpu.VMEM((128,), _i32),
            pltpu.VMEM((176,), _i32),
        ],
    )
    def pools(x_hbm, i0t_hbm, i1t_hbm, permt_hbm, xb_hbm,
              p1_hbm, p0_hbm, dg_hbm,
              p1buf, p0buf, dbuf, xbuf, xtbuf, i0b, i1b, ptb, idb, xbb):
        wid = _wid()

        def chunk_body(ci, _):
            n0 = pl.multiple_of(wid * (2 * SN) + ci * SN, 8)
            pltpu.sync_copy(xb_hbm.at[pl.ds(n0, 176)], xbb)
            r0 = _sread(xbb, 0)
            r1 = _sread(xbb, SN)
            base = r0 & ~7
            nb = (r1 - base + 127) // 128

            _zero_rows(p1buf, SN + 1)
            _zero_rows(p0buf, SN + 1)
            _zero_rows(dbuf, SN + 1)

            def batch_body(bi, _):
                s = pl.multiple_of(base + bi * 128, 8)
                pltpu.sync_copy(i0t_hbm.at[pl.ds(s, 128)], i0b.at[pl.ds(0, 128)])
                pltpu.sync_copy(i1t_hbm.at[pl.ds(s, 128)], i1b.at[pl.ds(0, 128)])
                pltpu.sync_copy(permt_hbm.at[pl.ds(s, 128)], ptb)
                for j in range(128 // L):
                    v = s + L * j + lax.iota(_i32, L)
                    idb[pl.ds(L * j, L)] = jnp.minimum(v, nnz - 1)
                pltpu.sync_copy(x_hbm.at[idb], xbuf)
                pltpu.sync_copy(x_hbm.at[ptb], xtbuf)

                def row_body(k, _):
                    t0 = _sread(i0b, k)
                    t1 = _sread(i1b, k)
                    off = t0 - n0
                    valid = (off >= 0) & (off < SN)
                    offc = jnp.where(valid, off, SN)
                    _row_add(p1buf, offc, xbuf, k)
                    _row_add(p0buf, offc, xtbuf, k)

                    @pl.when(valid & (t0 == t1))
                    def _():
                        _row_add(dbuf, offc, xbuf, k)

                    return 0

                lax.fori_loop(0, 128, row_body, 0)
                return 0

            lax.fori_loop(0, nb, batch_body, 0)

            full = n0 + SN <= N
            part = n0 == (N // SN) * SN

            @pl.when(full)
            def _():
                pltpu.sync_copy(p1buf.at[pl.ds(0, SN)], p1_hbm.at[pl.ds(n0, SN)])
                pltpu.sync_copy(p0buf.at[pl.ds(0, SN)], p0_hbm.at[pl.ds(n0, SN)])
                pltpu.sync_copy(dbuf.at[pl.ds(0, SN)], dg_hbm.at[pl.ds(n0, SN)])

            @pl.when(part)
            def _():
                rem = N - (N // SN) * SN
                pltpu.sync_copy(p1buf.at[pl.ds(0, rem)], p1_hbm.at[pl.ds(n0, rem)])
                pltpu.sync_copy(p0buf.at[pl.ds(0, rem)], p0_hbm.at[pl.ds(n0, rem)])
                pltpu.sync_copy(dbuf.at[pl.ds(0, rem)], dg_hbm.at[pl.ds(n0, rem)])

            return 0

        lax.fori_loop(0, 2, chunk_body, 0)

    return pools


def _make_x5_kernel():
    mesh = plsc.VectorSubcoreMesh(core_axis_name="c", subcore_axis_name="s")
    SN = 320

    @functools.partial(
        pl.kernel,
        out_type=jax.ShapeDtypeStruct((N, D), _f32),
        mesh=mesh,
        scratch_types=[
            pltpu.VMEM((SN + 1, D), _f32),
            pltpu.VMEM((128, D), _f32),
            pltpu.VMEM((128,), _i32),
            pltpu.VMEM((144,), _i32),
            pltpu.VMEM((SN + 16,), _i32),
        ],
    )
    def x5k(p0_hbm, adst_hbm, asrc_hbm, ap_hbm, x5_hbm,
            x5buf, gbuf, adb, asb, apb):
        wid = _wid()
        n0 = pl.multiple_of(wid * SN, 8)
        pltpu.sync_copy(ap_hbm.at[pl.ds(n0, SN + 16)], apb)
        e0 = _sread(apb, 0)
        e1 = _sread(apb, SN)
        base = e0 & ~7
        nb = (e1 - base + 127) // 128

        _zero_rows(x5buf, SN + 1)

        def batch_body(bi, _):
            s = pl.multiple_of(base + bi * 128, 8)
            pltpu.sync_copy(adst_hbm.at[pl.ds(s, 128)], adb)
            pltpu.sync_copy(asrc_hbm.at[pl.ds(s, 128)], asb.at[pl.ds(0, 128)])
            pltpu.sync_copy(p0_hbm.at[adb], gbuf)

            def edge_body(k, _):
                off = _sread(asb, k) - n0
                valid = (off >= 0) & (off < SN)
                offc = jnp.where(valid, off, SN)
                _row_add(x5buf, offc, gbuf, k)
                return 0

            lax.fori_loop(0, 128, edge_body, 0)
            return 0

        lax.fori_loop(0, nb, batch_body, 0)

        full = n0 + SN <= N
        part = n0 == (N // SN) * SN

        @pl.when(full)
        def _():
            pltpu.sync_copy(x5buf.at[pl.ds(0, SN)], x5_hbm.at[pl.ds(n0, SN)])

        @pl.when(part)
        def _():
            rem = N - (N // SN) * SN
            pltpu.sync_copy(x5buf.at[pl.ds(0, rem)], x5_hbm.at[pl.ds(n0, rem)])

    return x5k


def _make_out_kernel(nnz):
    mesh = plsc.VectorSubcoreMesh(core_axis_name="c", subcore_axis_name="s")
    SR = 256
    nch_total = (nnz + SR - 1) // SR
    last_c = nch_total - 1
    lastv = nnz - last_c * SR
    base_nch = nch_total // NT
    extra = nch_total - base_nch * NT

    @functools.partial(
        pl.kernel,
        out_type=jax.ShapeDtypeStruct((nnz, D), _f32),
        mesh=mesh,
        scratch_types=[
            pltpu.VMEM((SR + 1, D), _f32),
            pltpu.VMEM((SR, D), _f32),
            pltpu.VMEM((128, D), _f32),
            pltpu.VMEM((128,), _i32),
            pltpu.VMEM((128,), _i32),
            pltpu.VMEM((144,), _i32),
            pltpu.VMEM((SR + 16,), _i32),
        ],
    )
    def outk(g0_hbm, g1_hbm, y1_hbm, i0_hbm, i1_hbm, mpo_hbm, mps_hbm, pp_hbm,
             out_hbm, outbuf, bbuf, ybuf, ib, pob, psb, ppb):
        wid = _wid()
        nch = base_nch + jnp.where(wid < extra, 1, 0)

        def chunk_body(ci, _):
            c = wid + NT * ci
            ar = pl.multiple_of(c * SR, 8)
            pltpu.sync_copy(pp_hbm.at[pl.ds(ar, SR + 16)], ppb)
            p0 = _sread(ppb, 0)
            p1 = _sread(ppb, SR)
            base = p0 & ~7
            nb = (p1 - base + 127) // 128

            for h in range(SR // 128):
                pltpu.sync_copy(i0_hbm.at[pl.ds(ar + h * 128, 128)], ib)
                pltpu.sync_copy(g0_hbm.at[ib], outbuf.at[pl.ds(h * 128, 128)])
            for h in range(SR // 128):
                pltpu.sync_copy(i1_hbm.at[pl.ds(ar + h * 128, 128)], ib)
                pltpu.sync_copy(g1_hbm.at[ib], bbuf.at[pl.ds(h * 128, 128)])

            def init_body(r, _):
                _row_add(outbuf, r, bbuf, r)
                return 0

            lax.fori_loop(0, SR, init_body, 0)

            def batch_body(bi, _):
                s = pl.multiple_of(base + bi * 128, 8)
                pltpu.sync_copy(mpo_hbm.at[pl.ds(s, 128)], pob)
                pltpu.sync_copy(mps_hbm.at[pl.ds(s, 128)], psb.at[pl.ds(0, 128)])
                pltpu.sync_copy(y1_hbm.at[pob], ybuf)

                def pair_body(k, _):
                    off = _sread(psb, k) - ar
                    valid = (off >= 0) & (off < SR)
                    offc = jnp.where(valid, off, SR)
                    _row_add(outbuf, offc, ybuf, k)
                    return 0

                lax.fori_loop(0, 128, pair_body, 0)
                return 0

            lax.fori_loop(0, nb, batch_body, 0)

            @pl.when(c != last_c)
            def _():
                pltpu.sync_copy(outbuf.at[pl.ds(0, SR)], out_hbm.at[pl.ds(ar, SR)])

            @pl.when(c == last_c)
            def _():
                pltpu.sync_copy(outbuf.at[pl.ds(0, lastv)],
                                out_hbm.at[pl.ds(last_c * SR, lastv)])

            return 0

        lax.fori_loop(0, nch, chunk_body, 0)

    return outk


def kernel(x_values, W, b, x_indices, a_indices, mp_src, mp_out):
    nnz = x_values.shape[0]
    i0 = x_indices[0].astype(_i32)
    i1 = x_indices[1].astype(_i32)
    a_src = a_indices[0].astype(_i32)
    a_dst = a_indices[1].astype(_i32)
    mps = mp_src.astype(_i32)
    mpo = mp_out.astype(_i32)

    code = i0 * N + i1
    permt = jnp.searchsorted(code, i1 * N + i0).astype(_i32)
    xb = jnp.searchsorted(i0, jnp.arange(N + 1, dtype=_i32)).astype(_i32)
    ap = jnp.searchsorted(a_src, jnp.arange(N + 1, dtype=_i32)).astype(_i32)
    pp = jnp.searchsorted(mps, jnp.arange(nnz + 1, dtype=_i32)).astype(_i32)

    i0t = jnp.pad(i0, (0, PAD), constant_values=-1)
    i1t = jnp.pad(i1, (0, PAD), constant_values=-2)
    i0g = jnp.pad(i0, (0, PAD), constant_values=0)
    i1g = jnp.pad(i1, (0, PAD), constant_values=0)
    permtg = jnp.pad(permt, (0, PAD), constant_values=0)
    mpog = jnp.pad(mpo, (0, PAD), constant_values=0)
    mpst = jnp.pad(mps, (0, PAD), constant_values=-1)
    adg = jnp.pad(a_dst, (0, PAD), constant_values=0)
    ast = jnp.pad(a_src, (0, PAD), constant_values=-1)
    xb_p = jnp.pad(xb, (0, 512), constant_values=nnz)
    ap_p = jnp.pad(ap, (0, 512), constant_values=a_src.shape[0])
    pp_p = jnp.pad(pp, (0, 512), constant_values=mps.shape[0])

    w1, w2, w3, w4, w5, w6 = (W[D * k:D * (k + 1)] for k in range(6))
    b2d = b.reshape(1, D)

    y1 = _tc_matmul(x_values, w1)

    pool1, pool0, diag = _make_pools_kernel(nnz)(
        x_values, i0t, i1t, permtg, xb_p)

    x5 = _make_x5_kernel()(pool0, adg, ast, ap_p)

    g0, g1 = _tc_node_matmul(diag, pool1, pool0, x5, w2, w3, w4, w5, w6, b2d)

    out = _make_out_kernel(nnz)(g0, g1, y1, i0g, i1g, mpog, mpst, pp_p)
    return out


# speedup vs baseline: 25.9191x; 25.9191x over previous
import functools

import jax
import jax.numpy as jnp
from jax import lax
from jax.experimental import pallas as pl
from jax.experimental.pallas import tpu as pltpu
from jax.experimental.pallas import tpu_sc as plsc

N = 10000
D = 128
L = 16
NT = 32
PAD = 640

_f32 = jnp.float32
_i32 = jnp.int32


def _wid():
    return lax.axis_index("s") * 2 + lax.axis_index("c")


def _sread(ref, idx):
    return ref[pl.ds(idx, L)][0]


def _row_add(dst, dst_row, src, src_row):
    for j in range(D // L):
        v = src[src_row, pl.ds(L * j, L)]
        plsc.addupdate(dst.at[dst_row, pl.ds(L * j, L)], v)


def _zero_rows(buf, nrows):
    z = jnp.zeros((L,), _f32)

    def body(r, _):
        for j in range(D // L):
            buf[r, pl.ds(L * j, L)] = z
        return 0

    lax.fori_loop(0, nrows, body, 0)


def _mm_body(x_ref, w_ref, o_ref):
    o_ref[...] = jnp.dot(x_ref[...], w_ref[...], preferred_element_type=_f32)


def _tc_matmul(x, w):
    nnz = x.shape[0]
    br = 2048
    g = (nnz + br - 1) // br
    return pl.pallas_call(
        _mm_body,
        grid=(g,),
        in_specs=[
            pl.BlockSpec((br, D), lambda i: (i, 0)),
            pl.BlockSpec((D, D), lambda i: (0, 0)),
        ],
        out_specs=pl.BlockSpec((br, D), lambda i: (i, 0)),
        out_shape=jax.ShapeDtypeStruct((nnz, D), _f32),
    )(x, w)


def _node_mm_body(d_ref, p1_ref, p0_ref, x5_ref, w2, w3, w4, w5, w6, b_ref,
                  g0_ref, g1_ref):
    dd = d_ref[...]
    g0_ref[...] = (jnp.dot(dd, w2[...], preferred_element_type=_f32)
                   + jnp.dot(p0_ref[...], w5[...], preferred_element_type=_f32)
                   + jnp.dot(x5_ref[...], w6[...], preferred_element_type=_f32)
                   + b_ref[...])
    g1_ref[...] = (jnp.dot(dd, w3[...], preferred_element_type=_f32)
                   + jnp.dot(p1_ref[...], w4[...], preferred_element_type=_f32))


def _tc_node_matmul(diag, pool1, pool0, x5, w2, w3, w4, w5, w6, b2d):
    br = 1000
    g = N // br
    full = pl.BlockSpec((D, D), lambda i: (0, 0))
    blk = pl.BlockSpec((br, D), lambda i: (i, 0))
    return pl.pallas_call(
        _node_mm_body,
        grid=(g,),
        in_specs=[blk, blk, blk, blk, full, full, full, full, full,
                  pl.BlockSpec((1, D), lambda i: (0, 0))],
        out_specs=[blk, blk],
        out_shape=[jax.ShapeDtypeStruct((N, D), _f32),
                   jax.ShapeDtypeStruct((N, D), _f32)],
    )(diag, pool1, pool0, x5, w2, w3, w4, w5, w6, b2d)


def _make_pools_kernel(nnz):
    mesh = plsc.VectorSubcoreMesh(core_axis_name="c", subcore_axis_name="s")
    SN = 120
    NS = 10016
    ZR = 624

    @functools.partial(
        pl.kernel,
        out_type=[jax.ShapeDtypeStruct((N, D), _f32),
                  jax.ShapeDtypeStruct((2, N, D), _f32),
                  jax.ShapeDtypeStruct((N, D), _f32)],
        mesh=mesh,
        scratch_types=[
            pltpu.VMEM((SN + 1, D), _f32),
            pltpu.VMEM((SN + 1, D), _f32),
            pltpu.VMEM((128, D), _f32),
            pltpu.VMEM((144,), _i32),
            pltpu.VMEM((144,), _i32),
            pltpu.VMEM((128,), _i32),
            pltpu.VMEM((128,), _i32),
            pltpu.VMEM((112,), _i32),
            pltpu.VMEM_SHARED((NS, D), _f32),
        ],
    )
    def pools(x_hbm, i0t_hbm, i1t_hbm, xbs_hbm,
              p1_hbm, p0p_hbm, dg_hbm,
              p1buf, dbuf, xbuf, i0b, i1b, i1c, idb, xbsb, spool):
        cid = lax.axis_index("c")
        sid = lax.axis_index("s")
        wid = sid * 2 + cid

        pltpu.sync_copy(xbs_hbm, xbsb)

        _zero_rows(xbuf, 128)
        z0 = pl.multiple_of(sid * ZR, 8)
        for h in range(4):
            pltpu.sync_copy(xbuf.at[pl.ds(0, 128)],
                            spool.at[pl.ds(z0 + 128 * h, 128)])

        @pl.when(sid < 15)
        def _():
            pltpu.sync_copy(xbuf.at[pl.ds(0, 112)],
                            spool.at[pl.ds(z0 + 512, 112)])

        @pl.when(sid == 15)
        def _():
            pltpu.sync_copy(xbuf.at[pl.ds(0, 128)],
                            spool.at[pl.ds(z0 + 512, 128)])

        plsc.subcore_barrier()

        def chunk_body(ci, _):
            n0 = pl.multiple_of(wid * (3 * SN) + ci * SN, 8)
            r0 = _sread(xbsb, 3 * wid + ci)
            r1 = _sread(xbsb, 3 * wid + ci + 1)
            base = r0 & ~7
            nb = (r1 - base + 127) // 128

            _zero_rows(p1buf, SN + 1)
            _zero_rows(dbuf, SN + 1)

            def batch_body(bi, _):
                s = pl.multiple_of(base + bi * 128, 8)
                pltpu.sync_copy(i0t_hbm.at[pl.ds(s, 128)], i0b.at[pl.ds(0, 128)])
                pltpu.sync_copy(i1t_hbm.at[pl.ds(s, 128)], i1b.at[pl.ds(0, 128)])
                for j in range(128 // L):
                    v = s + L * j + lax.iota(_i32, L)
                    idb[pl.ds(L * j, L)] = jnp.minimum(v, nnz - 1)
                pltpu.sync_copy(x_hbm.at[idb], xbuf)
                for j in range(128 // L):
                    pos = s + L * j + lax.iota(_i32, L)
                    okp = (pos >= r0) & (pos < r1)
                    i1v = i1b[pl.ds(L * j, L)]
                    i1c[pl.ds(L * j, L)] = jnp.where(okp, i1v, N)
                pltpu.sync_copy(xbuf, spool.at[i1c], add=True)

                def row_body(k, _):
                    t0 = _sread(i0b, k)
                    t1 = _sread(i1b, k)
                    off = t0 - n0
                    valid = (off >= 0) & (off < SN)
                    offc = jnp.where(valid, off, SN)
                    _row_add(p1buf, offc, xbuf, k)

                    @pl.when(valid & (t0 == t1))
                    def _():
                        _row_add(dbuf, offc, xbuf, k)

                    return 0

                lax.fori_loop(0, 128, row_body, 0)
                return 0

            lax.fori_loop(0, nb, batch_body, 0)

            full = n0 + SN <= N
            part = n0 == (N // SN) * SN

            @pl.when(full)
            def _():
                pltpu.sync_copy(p1buf.at[pl.ds(0, SN)], p1_hbm.at[pl.ds(n0, SN)])
                pltpu.sync_copy(dbuf.at[pl.ds(0, SN)], dg_hbm.at[pl.ds(n0, SN)])

            @pl.when(part)
            def _():
                rem = N - (N // SN) * SN
                pltpu.sync_copy(p1buf.at[pl.ds(0, rem)], p1_hbm.at[pl.ds(n0, rem)])
                pltpu.sync_copy(dbuf.at[pl.ds(0, rem)], dg_hbm.at[pl.ds(n0, rem)])

            return 0

        lax.fori_loop(0, 3, chunk_body, 0)

        plsc.subcore_barrier()
        w0 = pl.multiple_of(sid * ZR, 8)

        @pl.when(sid < 15)
        def _():
            pltpu.sync_copy(spool.at[pl.ds(w0, ZR)],
                            p0p_hbm.at[cid, pl.ds(w0, ZR)])

        @pl.when(sid == 15)
        def _():
            pltpu.sync_copy(spool.at[pl.ds(w0, 640)],
                            p0p_hbm.at[cid, pl.ds(w0, 640)])

    return pools


def _add_body(a_ref, b_ref, o_ref):
    o_ref[...] = a_ref[...] + b_ref[...]


def _tc_add(a, b):
    br = 1000
    blk = pl.BlockSpec((br, D), lambda i: (i, 0))
    return pl.pallas_call(
        _add_body,
        grid=(N // br,),
        in_specs=[blk, blk],
        out_specs=blk,
        out_shape=jax.ShapeDtypeStruct((N, D), _f32),
    )(a, b)


def _make_x5_kernel():
    mesh = plsc.VectorSubcoreMesh(core_axis_name="c", subcore_axis_name="s")
    SN = 320

    @functools.partial(
        pl.kernel,
        out_type=jax.ShapeDtypeStruct((N, D), _f32),
        mesh=mesh,
        scratch_types=[
            pltpu.VMEM((SN + 1, D), _f32),
            pltpu.VMEM((128, D), _f32),
            pltpu.VMEM((128,), _i32),
            pltpu.VMEM((144,), _i32),
            pltpu.VMEM((48,), _i32),
        ],
    )
    def x5k(p0_hbm, adst_hbm, asrc_hbm, ap_hbm, x5_hbm,
            x5buf, gbuf, adb, asb, apb):
        wid = _wid()
        n0 = pl.multiple_of(wid * SN, 8)
        pltpu.sync_copy(ap_hbm, apb)
        e0 = _sread(apb, wid)
        e1 = _sread(apb, wid + 1)
        base = e0 & ~7
        nb = (e1 - base + 127) // 128

        _zero_rows(x5buf, SN + 1)

        def batch_body(bi, _):
            s = pl.multiple_of(base + bi * 128, 8)
            pltpu.sync_copy(adst_hbm.at[pl.ds(s, 128)], adb)
            pltpu.sync_copy(asrc_hbm.at[pl.ds(s, 128)], asb.at[pl.ds(0, 128)])
            pltpu.sync_copy(p0_hbm.at[adb], gbuf)

            def edge_body(k, _):
                off = _sread(asb, k) - n0
                valid = (off >= 0) & (off < SN)
                offc = jnp.where(valid, off, SN)
                _row_add(x5buf, offc, gbuf, k)
                return 0

            lax.fori_loop(0, 128, edge_body, 0)
            return 0

        lax.fori_loop(0, nb, batch_body, 0)

        full = n0 + SN <= N
        part = n0 == (N // SN) * SN

        @pl.when(full)
        def _():
            pltpu.sync_copy(x5buf.at[pl.ds(0, SN)], x5_hbm.at[pl.ds(n0, SN)])

        @pl.when(part)
        def _():
            rem = N - (N // SN) * SN
            pltpu.sync_copy(x5buf.at[pl.ds(0, rem)], x5_hbm.at[pl.ds(n0, rem)])

    return x5k


def _make_out_kernel(nnz):
    mesh = plsc.VectorSubcoreMesh(core_axis_name="c", subcore_axis_name="s")
    SR = 256
    nch_total = (nnz + SR - 1) // SR
    last_c = nch_total - 1
    lastv = nnz - last_c * SR
    base_nch = nch_total // NT
    extra = nch_total - base_nch * NT

    @functools.partial(
        pl.kernel,
        out_type=jax.ShapeDtypeStruct((nnz, D), _f32),
        mesh=mesh,
        scratch_types=[
            pltpu.VMEM((SR + 1, D), _f32),
            pltpu.VMEM((SR, D), _f32),
            pltpu.VMEM((128, D), _f32),
            pltpu.VMEM((128,), _i32),
            pltpu.VMEM((128,), _i32),
            pltpu.VMEM((144,), _i32),
            pltpu.VMEM((688,), _i32),
        ],
    )
    def outk(g0_hbm, g1_hbm, y1_hbm, i0_hbm, i1_hbm, mpo_hbm, mps_hbm, pp_hbm,
             out_hbm, outbuf, bbuf, ybuf, ib, pob, psb, ppb):
        wid = _wid()
        nch = base_nch + jnp.where(wid < extra, 1, 0)
        pltpu.sync_copy(pp_hbm, ppb)

        def chunk_body(ci, _):
            c = wid + NT * ci
            ar = pl.multiple_of(c * SR, 8)
            p0 = _sread(ppb, c)
            p1 = _sread(ppb, c + 1)
            base = p0 & ~7
            nb = (p1 - base + 127) // 128

            for h in range(SR // 128):
                pltpu.sync_copy(i0_hbm.at[pl.ds(ar + h * 128, 128)], ib)
                pltpu.sync_copy(g0_hbm.at[ib], outbuf.at[pl.ds(h * 128, 128)])
            for h in range(SR // 128):
                pltpu.sync_copy(i1_hbm.at[pl.ds(ar + h * 128, 128)], ib)
                pltpu.sync_copy(g1_hbm.at[ib], bbuf.at[pl.ds(h * 128, 128)])

            def init_body(r, _):
                _row_add(outbuf, r, bbuf, r)
                return 0

            lax.fori_loop(0, SR, init_body, 0)

            def batch_body(bi, _):
                s = pl.multiple_of(base + bi * 128, 8)
                pltpu.sync_copy(mpo_hbm.at[pl.ds(s, 128)], pob)
                pltpu.sync_copy(mps_hbm.at[pl.ds(s, 128)], psb.at[pl.ds(0, 128)])
                pltpu.sync_copy(y1_hbm.at[pob], ybuf)

                def pair_body(k, _):
                    off = _sread(psb, k) - ar
                    valid = (off >= 0) & (off < SR)
                    offc = jnp.where(valid, off, SR)
                    _row_add(outbuf, offc, ybuf, k)
                    return 0

                lax.fori_loop(0, 128, pair_body, 0)
                return 0

            lax.fori_loop(0, nb, batch_body, 0)

            @pl.when(c != last_c)
            def _():
                pltpu.sync_copy(outbuf.at[pl.ds(0, SR)], out_hbm.at[pl.ds(ar, SR)])

            @pl.when(c == last_c)
            def _():
                pltpu.sync_copy(outbuf.at[pl.ds(0, lastv)],
                                out_hbm.at[pl.ds(last_c * SR, lastv)])

            return 0

        lax.fori_loop(0, nch, chunk_body, 0)

    return outk


def kernel(x_values, W, b, x_indices, a_indices, mp_src, mp_out):
    nnz = x_values.shape[0]
    i0 = x_indices[0].astype(_i32)
    i1 = x_indices[1].astype(_i32)
    a_src = a_indices[0].astype(_i32)
    a_dst = a_indices[1].astype(_i32)
    mps = mp_src.astype(_i32)
    mpo = mp_out.astype(_i32)

    SRK4 = 256
    nch = (nnz + SRK4 - 1) // SRK4
    pps = jnp.searchsorted(mps, jnp.arange(0, (nch + 1) * SRK4, SRK4,
                                           dtype=_i32)).astype(_i32)
    pps = jnp.pad(pps, (0, 688 - pps.shape[0]))
    xbs = jnp.searchsorted(i0, jnp.arange(0, 11521, 120, dtype=_i32)).astype(_i32)
    xbs = jnp.pad(xbs, (0, 112 - xbs.shape[0]), constant_values=nnz)
    aps = jnp.searchsorted(a_src, jnp.arange(0, 10241, 320, dtype=_i32)).astype(_i32)
    aps = jnp.pad(aps, (0, 48 - aps.shape[0]), constant_values=a_src.shape[0])

    i0t = jnp.pad(i0, (0, PAD), constant_values=-1)
    i1t = jnp.pad(i1, (0, PAD), constant_values=-2)
    i0g = jnp.pad(i0, (0, PAD), constant_values=0)
    i1g = jnp.pad(i1, (0, PAD), constant_values=0)
    mpog = jnp.pad(mpo, (0, PAD), constant_values=0)
    mpst = jnp.pad(mps, (0, PAD), constant_values=-1)
    adg = jnp.pad(a_dst, (0, PAD), constant_values=0)
    ast = jnp.pad(a_src, (0, PAD), constant_values=-1)

    w1, w2, w3, w4, w5, w6 = (W[D * k:D * (k + 1)] for k in range(6))
    b2d = b.reshape(1, D)

    y1 = _tc_matmul(x_values, w1)

    pool1, p0parts, diag = _make_pools_kernel(nnz)(x_values, i0t, i1t, xbs)
    pool0 = _tc_add(p0parts[0], p0parts[1])

    x5 = _make_x5_kernel()(pool0, adg, ast, aps)

    g0, g1 = _tc_node_matmul(diag, pool1, pool0, x5, w2, w3, w4, w5, w6, b2d)

    out = _make_out_kernel(nnz)(g0, g1, y1, i0g, i1g, mpog, mpst, pps)
    return out
